# baseline (speedup 1.0000x reference)
import functools

import jax
import jax.numpy as jnp
from jax import lax
from jax.experimental import pallas as pl
from jax.experimental.pallas import tpu as pltpu
from jax.experimental.pallas import tpu_sc as plsc

N = 10000
D = 128
E = 320000

NC = 2
NS = 16
NW = NC * NS
EPW = E // NW
CH = 80
NCHUNK = EPW // CH
N_PAD = 10240
RPS = N_PAD // NS

_MESH = dict(core_axis_name="c", subcore_axis_name="s", num_cores=NC,
             num_subcores=NS)


NB = 2
NG = NCHUNK // NB


def _seg_body(gidx_hbm, dst_hbm, table_hbm, acc_out,
              gidx_v, dst_v, rows_v, acc_sh,
              gsem0, gsem1, ssem0, ssem1, isg0, isg1, isd0, isd1):
  gsem = (gsem0, gsem1)
  ssem = (ssem0, ssem1)
  isg = (isg0, isg1)
  isd = (isd0, isd1)
  c = lax.axis_index("c")
  s = lax.axis_index("s")
  wid = c * NS + s
  row0 = s * RPS

  def _z(i, _):
    def _zc(j, _):
      rows_v[0, i, pl.ds(j * 16, 16)] = jnp.zeros((16,), jnp.float32)
      return 0
    return lax.fori_loop(0, D // 16, _zc, 0)
  lax.fori_loop(0, CH, _z, 0)
  for k in range(RPS // CH):
    pltpu.sync_copy(rows_v.at[0], acc_sh.at[pl.ds(row0 + k * CH, CH), :])
  plsc.subcore_barrier()

  ebase = wid * EPW

  def _group(g, _):
    idesc = []
    for b in range(NB):
      off = ebase + (g * NB + b) * CH
      idesc.append((
          pltpu.async_copy(gidx_hbm.at[pl.ds(off, CH)], gidx_v.at[b], isg[b]),
          pltpu.async_copy(dst_hbm.at[pl.ds(off, CH)], dst_v.at[b], isd[b]),
      ))
    gdesc = []
    for b in range(NB):
      idesc[b][0].wait()
      gdesc.append(
          pltpu.async_copy(table_hbm.at[gidx_v.at[b]], rows_v.at[b], gsem[b]))
    sdesc = []
    for b in range(NB):
      gdesc[b].wait()
      idesc[b][1].wait()
      sdesc.append(
          pltpu.async_copy(rows_v.at[b], acc_sh.at[dst_v.at[b]], ssem[b],
                           add=True))
    for b in range(NB):
      sdesc[b].wait()
    return 0
  lax.fori_loop(0, NG, _group, 0)
  for q in range(NG * NB, NCHUNK):
    off = ebase + q * CH
    pltpu.sync_copy(gidx_hbm.at[pl.ds(off, CH)], gidx_v.at[0])
    pltpu.sync_copy(dst_hbm.at[pl.ds(off, CH)], dst_v.at[0])
    pltpu.async_copy(table_hbm.at[gidx_v.at[0]], rows_v.at[0], gsem[0]).wait()
    pltpu.sync_copy(rows_v.at[0], acc_sh.at[dst_v.at[0]], add=True)
  plsc.subcore_barrier()

  for k in range(RPS // CH):
    r = row0 + k * CH
    pltpu.sync_copy(acc_sh.at[pl.ds(r, CH), :], rows_v.at[0])
    pltpu.sync_copy(rows_v.at[0], acc_out.at[c, pl.ds(r, CH), :])


@functools.lru_cache(maxsize=None)
def _make_seg():
  return pl.kernel(
      _seg_body,
      out_type=[jax.ShapeDtypeStruct((NC, N_PAD, D), jnp.float32)],
      mesh=plsc.VectorSubcoreMesh(**_MESH),
      scratch_types=[
          pltpu.VMEM((NB, CH), jnp.int32),
          pltpu.VMEM((NB, CH), jnp.int32),
          pltpu.VMEM((NB, CH, D), jnp.float32),
          pltpu.VMEM_SHARED((N_PAD, D), jnp.float32),
          pltpu.SemaphoreType.DMA,
          pltpu.SemaphoreType.DMA,
          pltpu.SemaphoreType.DMA,
          pltpu.SemaphoreType.DMA,
          pltpu.SemaphoreType.DMA,
          pltpu.SemaphoreType.DMA,
          pltpu.SemaphoreType.DMA,
          pltpu.SemaphoreType.DMA,
      ],
      name="seg_sum",
  )


def _tc1_body(acc0, acc1, dacc0, dacc1, x, w1l_t, b1l, w1r_t, gamma, beta,
              rm, rv, h_out):
  deg = (dacc0[...] + dacc1[...])[:N, 0:1]
  agg = (acc0[...] + acc1[...])[:N] / jnp.maximum(deg, 1.0)
  out = (jnp.dot(agg, w1l_t[...], preferred_element_type=jnp.float32)
         + jnp.dot(x[...], w1r_t[...], preferred_element_type=jnp.float32)
         + b1l[...])
  nrm = jnp.sqrt(jnp.sum(out * out, axis=1, keepdims=True))
  out = out / jnp.maximum(nrm, 1e-12)
  out = jnp.maximum(out, 0.0)
  inv = lax.rsqrt(rv[...] + 1e-5)
  h_out[...] = (out - rm[...]) * inv * gamma[...] + beta[...]


def _tc2_body(acc0, acc1, dacc0, dacc1, h, w2l_t, b2l, w2r_t, wfc_t, bfc,
              embed_out, preds_out):
  deg = (dacc0[...] + dacc1[...])[:N, 0:1]
  agg = (acc0[...] + acc1[...])[:N] / jnp.maximum(deg, 1.0)
  out = (jnp.dot(agg, w2l_t[...], preferred_element_type=jnp.float32)
         + jnp.dot(h[...], w2r_t[...], preferred_element_type=jnp.float32)
         + b2l[...])
  nrm = jnp.sqrt(jnp.sum(out * out, axis=1, keepdims=True))
  embed = out / jnp.maximum(nrm, 1e-12)
  embed_out[...] = embed
  preds_out[...] = (jnp.dot(embed[:, :D // 2], wfc_t[...],
                            preferred_element_type=jnp.float32) + bfc[...])


def kernel(x, edge_index, W1l, b1l, W1r, gamma, beta, rm, rv, W2l, b2l,
           W2r, Wfc, bfc):
  src = edge_index[0]
  dst = edge_index[1]
  ones_table = jnp.ones((N, D), jnp.float32)

  seg = _make_seg()
  acc1p = seg(src, dst, x)[0]
  daccp = seg(src, dst, ones_table)[0]

  h = pl.pallas_call(
      _tc1_body,
      out_shape=jax.ShapeDtypeStruct((N, D), jnp.float32),
  )(acc1p[0], acc1p[1], daccp[0], daccp[1], x,
    W1l.T, b1l.reshape(1, D), W1r.T, gamma.reshape(1, D),
    beta.reshape(1, D), rm.reshape(1, D), rv.reshape(1, D))

  acc2p = seg(src, dst, h)[0]

  embed, preds = pl.pallas_call(
      _tc2_body,
      out_shape=[
          jax.ShapeDtypeStruct((N, D), jnp.float32),
          jax.ShapeDtypeStruct((N, 1), jnp.float32),
      ],
  )(acc2p[0], acc2p[1], daccp[0], daccp[1], h,
    W2l.T, b2l.reshape(1, D), W2r.T, Wfc.T, bfc.reshape(1, 1))

  return (preds[:, 0], embed)


# speedup vs baseline: 1.2864x; 1.2864x over previous
import functools

import jax
import jax.numpy as jnp
from jax import lax
from jax.experimental import pallas as pl
from jax.experimental.pallas import tpu as pltpu
from jax.experimental.pallas import tpu_sc as plsc

N = 10000
D = 128
E = 320000

NC = 2
NS = 16
NW = NC * NS
EPW = E // NW
CH = 80
NCHUNK = EPW // CH
N_PAD = 10240
RPS = N_PAD // NS

_MESH = dict(core_axis_name="c", subcore_axis_name="s", num_cores=NC,
             num_subcores=NS)


NB = 4
NG = NCHUNK // NB


def _seg_body(gidx_hbm, dst_hbm, table_hbm, acc_out,
              gidx_v, dst_v, rows_v, acc_sh, *sems):
  gsem = sems[0:NB]
  ssem = sems[NB:2 * NB]
  isg = sems[2 * NB:3 * NB]
  isd = sems[3 * NB:4 * NB]
  c = lax.axis_index("c")
  s = lax.axis_index("s")
  wid = c * NS + s
  row0 = s * RPS

  def _z(i, _):
    def _zc(j, _):
      rows_v[0, i, pl.ds(j * 16, 16)] = jnp.zeros((16,), jnp.float32)
      return 0
    return lax.fori_loop(0, D // 16, _zc, 0)
  lax.fori_loop(0, CH, _z, 0)
  for k in range(RPS // CH):
    pltpu.sync_copy(rows_v.at[0], acc_sh.at[pl.ds(row0 + k * CH, CH), :])
  plsc.subcore_barrier()

  ebase = wid * EPW

  def _group(g, _):
    idesc = []
    for b in range(NB):
      off = ebase + (g * NB + b) * CH

      @pl.when(g > 0)
      def _drain(b=b):
        pltpu.make_async_copy(rows_v.at[b], acc_sh.at[dst_v.at[b]],
                              ssem[b]).wait()

      idesc.append((
          pltpu.async_copy(gidx_hbm.at[pl.ds(off, CH)], gidx_v.at[b], isg[b]),
          pltpu.async_copy(dst_hbm.at[pl.ds(off, CH)], dst_v.at[b], isd[b]),
      ))
    gdesc = []
    for b in range(NB):
      idesc[b][0].wait()
      gdesc.append(
          pltpu.async_copy(table_hbm.at[gidx_v.at[b]], rows_v.at[b], gsem[b]))
    for b in range(NB):
      gdesc[b].wait()
      idesc[b][1].wait()
      pltpu.async_copy(rows_v.at[b], acc_sh.at[dst_v.at[b]], ssem[b],
                       add=True)
    return 0
  lax.fori_loop(0, NG, _group, 0)
  for b in range(NB):
    pltpu.make_async_copy(rows_v.at[b], acc_sh.at[dst_v.at[b]],
                          ssem[b]).wait()
  for q in range(NG * NB, NCHUNK):
    off = ebase + q * CH
    pltpu.sync_copy(gidx_hbm.at[pl.ds(off, CH)], gidx_v.at[0])
    pltpu.sync_copy(dst_hbm.at[pl.ds(off, CH)], dst_v.at[0])
    pltpu.async_copy(table_hbm.at[gidx_v.at[0]], rows_v.at[0], gsem[0]).wait()
    pltpu.sync_copy(rows_v.at[0], acc_sh.at[dst_v.at[0]], add=True)
  plsc.subcore_barrier()

  for k in range(RPS // CH):
    r = row0 + k * CH
    pltpu.sync_copy(acc_sh.at[pl.ds(r, CH), :], rows_v.at[0])
    pltpu.sync_copy(rows_v.at[0], acc_out.at[c, pl.ds(r, CH), :])


@functools.lru_cache(maxsize=None)
def _make_seg():
  return pl.kernel(
      _seg_body,
      out_type=[jax.ShapeDtypeStruct((NC, N_PAD, D), jnp.float32)],
      mesh=plsc.VectorSubcoreMesh(**_MESH),
      scratch_types=[
          pltpu.VMEM((NB, CH), jnp.int32),
          pltpu.VMEM((NB, CH), jnp.int32),
          pltpu.VMEM((NB, CH, D), jnp.float32),
          pltpu.VMEM_SHARED((N_PAD, D), jnp.float32),
      ] + [pltpu.SemaphoreType.DMA] * (4 * NB),
      name="seg_sum",
  )


def _tc1_body(acc0, acc1, dacc0, dacc1, x, w1l_t, b1l, w1r_t, gamma, beta,
              rm, rv, h_out):
  deg = (dacc0[...] + dacc1[...])[:N, 0:1]
  agg = (acc0[...] + acc1[...])[:N] / jnp.maximum(deg, 1.0)
  out = (jnp.dot(agg, w1l_t[...], preferred_element_type=jnp.float32)
         + jnp.dot(x[...], w1r_t[...], preferred_element_type=jnp.float32)
         + b1l[...])
  nrm = jnp.sqrt(jnp.sum(out * out, axis=1, keepdims=True))
  out = out / jnp.maximum(nrm, 1e-12)
  out = jnp.maximum(out, 0.0)
  inv = lax.rsqrt(rv[...] + 1e-5)
  h_out[...] = (out - rm[...]) * inv * gamma[...] + beta[...]


def _tc2_body(acc0, acc1, dacc0, dacc1, h, w2l_t, b2l, w2r_t, wfc_t, bfc,
              embed_out, preds_out):
  deg = (dacc0[...] + dacc1[...])[:N, 0:1]
  agg = (acc0[...] + acc1[...])[:N] / jnp.maximum(deg, 1.0)
  out = (jnp.dot(agg, w2l_t[...], preferred_element_type=jnp.float32)
         + jnp.dot(h[...], w2r_t[...], preferred_element_type=jnp.float32)
         + b2l[...])
  nrm = jnp.sqrt(jnp.sum(out * out, axis=1, keepdims=True))
  embed = out / jnp.maximum(nrm, 1e-12)
  embed_out[...] = embed
  preds_out[...] = (jnp.dot(embed[:, :D // 2], wfc_t[...],
                            preferred_element_type=jnp.float32) + bfc[...])


def kernel(x, edge_index, W1l, b1l, W1r, gamma, beta, rm, rv, W2l, b2l,
           W2r, Wfc, bfc):
  src = edge_index[0]
  dst = edge_index[1]
  ones_table = jnp.ones((N, D), jnp.float32)

  seg = _make_seg()
  acc1p = seg(src, dst, x)[0]
  daccp = seg(src, dst, ones_table)[0]

  h = pl.pallas_call(
      _tc1_body,
      out_shape=jax.ShapeDtypeStruct((N, D), jnp.float32),
  )(acc1p[0], acc1p[1], daccp[0], daccp[1], x,
    W1l.T, b1l.reshape(1, D), W1r.T, gamma.reshape(1, D),
    beta.reshape(1, D), rm.reshape(1, D), rv.reshape(1, D))

  acc2p = seg(src, dst, h)[0]

  embed, preds = pl.pallas_call(
      _tc2_body,
      out_shape=[
          jax.ShapeDtypeStruct((N, D), jnp.float32),
          jax.ShapeDtypeStruct((N, 1), jnp.float32),
      ],
  )(acc2p[0], acc2p[1], daccp[0], daccp[1], h,
    W2l.T, b2l.reshape(1, D), W2r.T, Wfc.T, bfc.reshape(1, 1))

  return (preds[:, 0], embed)
